# baseline (speedup 1.0000x reference)
import functools

import jax
import jax.numpy as jnp
from jax import lax
from jax.experimental import pallas as pl
from jax.experimental.pallas import tpu as pltpu
from jax.experimental.pallas import tpu_sc as plsc

_R = 32
_NV = _R * _R * _R
_B, _N = 8, 65536
_FC = 32
_GROUPS = 4
_CPG = _FC // _GROUPS
_QN = _N // _GROUPS
_CCH = 1024
_NQC = _QN // _CCH
_FCH = 4096
_NCH = _N // _FCH
_DCH = 4096
_NDC = _NV // _DCH
_STR = _NV // _GROUPS


def _make_sc_kernel():
  mesh = plsc.VectorSubcoreMesh(core_axis_name="c", subcore_axis_name="s")

  @functools.partial(
      pl.kernel,
      out_type=(
          jax.ShapeDtypeStruct((_B, _FC, _NV), jnp.float32),
          jax.ShapeDtypeStruct((_B, 3, _N), jnp.int32),
      ),
      mesh=mesh,
      scratch_types=[
          pltpu.VMEM((_NV,), jnp.float32),
          pltpu.VMEM((_NV,), jnp.float32),
          pltpu.VMEM((2, _FCH), jnp.float32),
          pltpu.VMEM((2, _FCH // 2), jnp.float32),
          pltpu.VMEM((2, _DCH), jnp.float32),
          pltpu.VMEM((2, 3, _CCH), jnp.float32),
          pltpu.VMEM((2, 3, _CCH), jnp.int32),
          pltpu.VMEM((2, _CCH // 2), jnp.float32),
          pltpu.VMEM_SHARED((_GROUPS, _N // 2), jnp.float32),
          pltpu.VMEM_SHARED((_GROUPS, _NV), jnp.float32),
          pltpu.HBM((_B * _GROUPS * _NV,), jnp.float32),
          pltpu.SemaphoreType.DMA((12,)),
      ],
      compiler_params=pltpu.CompilerParams(needs_layout_passes=False),
  )
  def vox_kernel(f_hbm, c_hbm, out_hbm, vox_hbm,
                 acc_v, inv_v, fbuf, ibuf, dbuf, cbuf, vbuf, pbuf,
                 idx_s, inv_s, part_h, sems):
    cid = lax.axis_index("c")
    sid = lax.axis_index("s")
    wid = cid * 16 + sid
    b = wid // _GROUPS
    g = wid % _GROUPS
    bl = b % _GROUPS

    half = jnp.float32(0.5)
    one = jnp.float32(1.0)
    zero16 = jnp.zeros((16,), jnp.float32)
    ones16 = jnp.ones((16,), jnp.float32)

    def axis_round(vals):
      t = jnp.clip(vals * _R, 0.0, _R - 1.0)
      i0 = t.astype(jnp.int32)
      frac = t - i0.astype(jnp.float32)
      up = jnp.where(frac > half, 1, 0) + jnp.where(
          jnp.logical_and(frac == half, (i0 & 1) == 1), 1, 0)
      return i0 + up

    def zero_acc():
      def zloop(j, _):
        for u in range(8):
          acc_v[pl.ds((j * 8 + u) * 16, 16)] = zero16
        return 0

      lax.fori_loop(0, _NV // 128, zloop, 0)

    zero_acc()
    qbase = pl.multiple_of(g * _QN, _QN)
    qbase2 = pl.multiple_of(g * (_QN // 2), _QN // 2)

    for slot in range(2):
      pltpu.async_copy(c_hbm.at[b, :, pl.ds(qbase + slot * _CCH, _CCH)],
                       cbuf.at[slot], sems.at[6 + slot])

    def p1_pair(kp, _):
      for slot in range(2):
        kc = kp * 2 + slot
        base = qbase + kc * _CCH
        base2 = qbase2 + kc * (_CCH // 2)
        pltpu.make_async_copy(c_hbm.at[b, :, pl.ds(base, _CCH)],
                              cbuf.at[slot], sems.at[6 + slot]).wait()

        @pl.when(kp >= 1)
        def _():
          pltpu.make_async_copy(vbuf.at[slot],
                                vox_hbm.at[b, :, pl.ds(base, _CCH)],
                                sems.at[8 + slot]).wait()
          pltpu.make_async_copy(pbuf.at[slot],
                                idx_s.at[bl, pl.ds(base2, _CCH // 2)],
                                sems.at[10 + slot]).wait()

        def p1v(i, _):
          off = i * 32
          flats = []
          for u in range(2):
            o = off + u * 16
            vx = axis_round(cbuf[slot, 0, pl.ds(o, 16)])
            vy = axis_round(cbuf[slot, 1, pl.ds(o, 16)])
            vz = axis_round(cbuf[slot, 2, pl.ds(o, 16)])
            vbuf[slot, 0, pl.ds(o, 16)] = vx
            vbuf[slot, 1, pl.ds(o, 16)] = vy
            vbuf[slot, 2, pl.ds(o, 16)] = vz
            flats.append(vx * (_R * _R) + vy * _R + vz)
          pbuf[slot, pl.ds(off // 2, 16)] = plsc.bitcast(
              flats[0] | (flats[1] << 16), jnp.float32)
          plsc.addupdate_scatter(acc_v, [flats[0]], ones16)
          plsc.addupdate_scatter(acc_v, [flats[1]], ones16)
          return 0

        lax.fori_loop(0, _CCH // 32, p1v, 0)
        pltpu.async_copy(vbuf.at[slot],
                         vox_hbm.at[b, :, pl.ds(base, _CCH)],
                         sems.at[8 + slot])
        pltpu.async_copy(pbuf.at[slot],
                         idx_s.at[bl, pl.ds(base2, _CCH // 2)],
                         sems.at[10 + slot])

        @pl.when(kc + 2 < _NQC)
        def _():
          pltpu.async_copy(
              c_hbm.at[b, :, pl.ds(qbase + (kc + 2) * _CCH, _CCH)],
              cbuf.at[slot], sems.at[6 + slot])
      return 0

    with jax.named_scope("p1"):
      lax.fori_loop(0, _NQC // 2, p1_pair, 0)

    for slot in range(2):
      pltpu.make_async_copy(vbuf.at[slot],
                            vox_hbm.at[b, :, pl.ds(slot * _CCH, _CCH)],
                            sems.at[8 + slot]).wait()
      pltpu.make_async_copy(pbuf.at[slot],
                            idx_s.at[bl, pl.ds(slot * _CCH // 2, _CCH // 2)],
                            sems.at[10 + slot]).wait()

    with jax.named_scope("pc"):
      pltpu.sync_copy(acc_v, part_h.at[pl.ds(pl.multiple_of((b * _GROUPS + g) * _NV, _NV), _NV)])
      zero_acc()
    plsc.subcore_barrier()

    sbase = pl.multiple_of(g * _STR, _STR)
    for j in range(_GROUPS):
      pltpu.sync_copy(
          part_h.at[pl.ds(
              pl.multiple_of((b * _GROUPS + j) * _NV + sbase, _STR), _STR)],
          inv_v.at[pl.ds(j * _STR, _STR)])

    def mloop(t, _):
      for u in range(4):
        o = (t * 4 + u) * 16
        v = (inv_v[pl.ds(o, 16)] + inv_v[pl.ds(_STR + o, 16)]
             + inv_v[pl.ds(2 * _STR + o, 16)]
             + inv_v[pl.ds(3 * _STR + o, 16)])
        inv_v[pl.ds(o, 16)] = one / jnp.maximum(v, one)
      return 0

    with jax.named_scope("mrg"):
      lax.fori_loop(0, _STR // 64, mloop, 0)
    pltpu.sync_copy(inv_v.at[pl.ds(0, _STR)],
                    inv_s.at[bl, pl.ds(sbase, _STR)])
    plsc.subcore_barrier()
    pltpu.sync_copy(inv_s.at[bl], inv_v)

    ch0 = g * _CPG
    for slot in range(2):
      pltpu.async_copy(f_hbm.at[b, ch0, pl.ds(slot * _FCH, _FCH)],
                       fbuf.at[slot], sems.at[slot])
      pltpu.async_copy(idx_s.at[bl, pl.ds(slot * _FCH // 2, _FCH // 2)],
                       ibuf.at[slot], sems.at[2 + slot])

    for cc in range(_CPG):
      ch = ch0 + cc

      def f_pair(kp, _):
        for slot in range(2):
          kc = kp * 2 + slot
          fbase = kc * _FCH
          fbase2 = kc * (_FCH // 2)
          pltpu.make_async_copy(f_hbm.at[b, ch, pl.ds(fbase, _FCH)],
                                fbuf.at[slot], sems.at[slot]).wait()
          pltpu.make_async_copy(
              idx_s.at[bl, pl.ds(fbase2, _FCH // 2)],
              ibuf.at[slot], sems.at[2 + slot]).wait()

          def svec(i, _):
            for u in range(8):
              o = i * 256 + u * 32
              w = plsc.bitcast(ibuf[slot, pl.ds(o // 2, 16)], jnp.int32)
              iv0 = w & 0xFFFF
              iv1 = lax.shift_right_logical(w, 16)
              plsc.addupdate_scatter(acc_v, [iv0], fbuf[slot, pl.ds(o, 16)])
              plsc.addupdate_scatter(acc_v, [iv1],
                                     fbuf[slot, pl.ds(o + 16, 16)])
            return 0

          lax.fori_loop(0, _FCH // 256, svec, 0)

          nxt = fbase + 2 * _FCH

          @pl.when(nxt < _N)
          def _():
            pltpu.async_copy(f_hbm.at[b, ch, pl.ds(nxt, _FCH)],
                             fbuf.at[slot], sems.at[slot])
            pltpu.async_copy(idx_s.at[bl, pl.ds(fbase2 + _FCH, _FCH // 2)],
                             ibuf.at[slot], sems.at[2 + slot])
        return 0

      with jax.named_scope(f"scat{cc}"):
        lax.fori_loop(0, _NCH // 2, f_pair, 0)

      if cc + 1 < _CPG:
        for slot in range(2):
          pltpu.async_copy(f_hbm.at[b, ch + 1, pl.ds(slot * _FCH, _FCH)],
                           fbuf.at[slot], sems.at[slot])
          pltpu.async_copy(idx_s.at[bl, pl.ds(slot * _FCH // 2, _FCH // 2)],
                           ibuf.at[slot], sems.at[2 + slot])

      def d_pair(mp, _):
        for slot in range(2):
          m = mp * 2 + slot
          dbase = m * _DCH

          @pl.when(mp >= 1)
          def _():
            pltpu.make_async_copy(dbuf.at[slot],
                                  out_hbm.at[b, ch, pl.ds(dbase, _DCH)],
                                  sems.at[4 + slot]).wait()

          def dvec(j, _):
            for u in range(8):
              o = (j * 8 + u) * 16
              a = acc_v[pl.ds(dbase + o, 16)]
              dbuf[slot, pl.ds(o, 16)] = a * inv_v[pl.ds(dbase + o, 16)]
              acc_v[pl.ds(dbase + o, 16)] = zero16
            return 0

          lax.fori_loop(0, _DCH // 128, dvec, 0)
          pltpu.async_copy(dbuf.at[slot],
                           out_hbm.at[b, ch, pl.ds(dbase, _DCH)],
                           sems.at[4 + slot])
        return 0

      with jax.named_scope(f"div{cc}"):
        lax.fori_loop(0, _NDC // 2, d_pair, 0)

      for slot in range(2):
        pltpu.make_async_copy(dbuf.at[slot],
                              out_hbm.at[b, ch, pl.ds(slot * _DCH, _DCH)],
                              sems.at[4 + slot]).wait()

  return vox_kernel


_vox_kernel = _make_sc_kernel()


@jax.jit
def kernel(features, coords):
  out_flat, vox_coords = _vox_kernel(features, coords)
  out = out_flat.reshape(_B, _FC, _R, _R, _R)
  return out, vox_coords


# speedup vs baseline: 1.7478x; 1.7478x over previous
import functools

import jax
import jax.numpy as jnp
from jax import lax
from jax.experimental import pallas as pl
from jax.experimental.pallas import tpu as pltpu
from jax.experimental.pallas import tpu_sc as plsc

_R = 32
_NV = _R * _R * _R
_B, _N = 8, 65536
_FC = 32
_GROUPS = 4
_CPG = _FC // _GROUPS
_QN = _N // _GROUPS
_CCH = 1024
_NQC = _QN // _CCH
_FCH = 4096
_NCH = _N // _FCH
_DCH = 4096
_NDC = _NV // _DCH
_STR = _NV // _GROUPS


def _make_sc_kernel():
  mesh = plsc.VectorSubcoreMesh(core_axis_name="c", subcore_axis_name="s")

  @functools.partial(
      pl.kernel,
      out_type=(
          jax.ShapeDtypeStruct((_B, _FC, _NV), jnp.float32),
          jax.ShapeDtypeStruct((_B, 3, _N), jnp.int32),
      ),
      mesh=mesh,
      scratch_types=[
          pltpu.VMEM((_NV,), jnp.float32),
          pltpu.VMEM((_NV,), jnp.float32),
          pltpu.VMEM((2, _FCH), jnp.float32),
          pltpu.VMEM((2, _FCH // 2), jnp.float32),
          pltpu.VMEM((2, _DCH), jnp.float32),
          pltpu.VMEM((2, 3, _CCH), jnp.float32),
          pltpu.VMEM((2, 3, _CCH), jnp.int32),
          pltpu.VMEM((2, _CCH // 2), jnp.float32),
          pltpu.VMEM_SHARED((_GROUPS, _N // 2), jnp.float32),
          pltpu.VMEM_SHARED((_GROUPS, _NV), jnp.float32),
          pltpu.HBM((_B * _GROUPS * _NV,), jnp.float32),
          pltpu.SemaphoreType.DMA((12,)),
      ],
      compiler_params=pltpu.CompilerParams(needs_layout_passes=False),
  )
  def vox_kernel(f_hbm, c_hbm, out_hbm, vox_hbm,
                 acc_v, inv_v, fbuf, ibuf, dbuf, cbuf, vbuf, pbuf,
                 idx_s, inv_s, part_h, sems):
    cid = lax.axis_index("c")
    sid = lax.axis_index("s")
    wid = cid * 16 + sid
    b = wid // _GROUPS
    g = wid % _GROUPS
    bl = b % _GROUPS

    half = jnp.float32(0.5)
    one = jnp.float32(1.0)
    zero16 = jnp.zeros((16,), jnp.float32)
    ones16 = jnp.ones((16,), jnp.float32)

    def axis_round(vals):
      t = jnp.clip(vals * _R, 0.0, _R - 1.0)
      i0 = t.astype(jnp.int32)
      frac = t - i0.astype(jnp.float32)
      up = jnp.where(frac > half, 1, 0) + jnp.where(
          jnp.logical_and(frac == half, (i0 & 1) == 1), 1, 0)
      return i0 + up

    def zero_acc():
      def zloop(j, _):
        for u in range(8):
          acc_v[pl.ds((j * 8 + u) * 16, 16)] = zero16
        return 0

      lax.fori_loop(0, _NV // 128, zloop, 0)

    zero_acc()
    qbase = pl.multiple_of(g * _QN, _QN)
    qbase2 = pl.multiple_of(g * (_QN // 2), _QN // 2)

    for slot in range(2):
      pltpu.async_copy(c_hbm.at[b, :, pl.ds(qbase + slot * _CCH, _CCH)],
                       cbuf.at[slot], sems.at[6 + slot])

    def p1_pair(kp, _):
      for slot in range(2):
        kc = kp * 2 + slot
        base = qbase + kc * _CCH
        base2 = qbase2 + kc * (_CCH // 2)
        pltpu.make_async_copy(c_hbm.at[b, :, pl.ds(base, _CCH)],
                              cbuf.at[slot], sems.at[6 + slot]).wait()

        @pl.when(kp >= 1)
        def _():
          pltpu.make_async_copy(vbuf.at[slot],
                                vox_hbm.at[b, :, pl.ds(base, _CCH)],
                                sems.at[8 + slot]).wait()
          pltpu.make_async_copy(pbuf.at[slot],
                                idx_s.at[bl, pl.ds(base2, _CCH // 2)],
                                sems.at[10 + slot]).wait()

        def p1v(i, _):
          off = i * 32
          flats = []
          for u in range(2):
            o = off + u * 16
            vx = axis_round(cbuf[slot, 0, pl.ds(o, 16)])
            vy = axis_round(cbuf[slot, 1, pl.ds(o, 16)])
            vz = axis_round(cbuf[slot, 2, pl.ds(o, 16)])
            vbuf[slot, 0, pl.ds(o, 16)] = vx
            vbuf[slot, 1, pl.ds(o, 16)] = vy
            vbuf[slot, 2, pl.ds(o, 16)] = vz
            flats.append(vx * (_R * _R) + vy * _R + vz)
          pbuf[slot, pl.ds(off // 2, 16)] = plsc.bitcast(
              flats[0] | (flats[1] << 16), jnp.float32)
          plsc.addupdate_scatter(acc_v, [flats[0]], ones16)
          plsc.addupdate_scatter(acc_v, [flats[1]], ones16)
          return 0

        lax.fori_loop(0, _CCH // 32, p1v, 0)
        pltpu.async_copy(vbuf.at[slot],
                         vox_hbm.at[b, :, pl.ds(base, _CCH)],
                         sems.at[8 + slot])
        pltpu.async_copy(pbuf.at[slot],
                         idx_s.at[bl, pl.ds(base2, _CCH // 2)],
                         sems.at[10 + slot])

        @pl.when(kc + 2 < _NQC)
        def _():
          pltpu.async_copy(
              c_hbm.at[b, :, pl.ds(qbase + (kc + 2) * _CCH, _CCH)],
              cbuf.at[slot], sems.at[6 + slot])
      return 0

    with jax.named_scope("p1"):
      lax.fori_loop(0, _NQC // 2, p1_pair, 0)

    for slot in range(2):
      pltpu.make_async_copy(vbuf.at[slot],
                            vox_hbm.at[b, :, pl.ds(slot * _CCH, _CCH)],
                            sems.at[8 + slot]).wait()
      pltpu.make_async_copy(pbuf.at[slot],
                            idx_s.at[bl, pl.ds(slot * _CCH // 2, _CCH // 2)],
                            sems.at[10 + slot]).wait()

    with jax.named_scope("pc"):
      pltpu.sync_copy(acc_v, part_h.at[pl.ds(pl.multiple_of((b * _GROUPS + g) * _NV, _NV), _NV)])
      zero_acc()
    plsc.subcore_barrier()

    sbase = pl.multiple_of(g * _STR, _STR)
    for j in range(_GROUPS):
      pltpu.sync_copy(
          part_h.at[pl.ds(
              pl.multiple_of((b * _GROUPS + j) * _NV + sbase, _STR), _STR)],
          inv_v.at[pl.ds(j * _STR, _STR)])

    def mloop(t, _):
      for u in range(4):
        o = (t * 4 + u) * 16
        v = (inv_v[pl.ds(o, 16)] + inv_v[pl.ds(_STR + o, 16)]
             + inv_v[pl.ds(2 * _STR + o, 16)]
             + inv_v[pl.ds(3 * _STR + o, 16)])
        inv_v[pl.ds(o, 16)] = one / jnp.maximum(v, one)
      return 0

    with jax.named_scope("mrg"):
      lax.fori_loop(0, _STR // 64, mloop, 0)
    pltpu.sync_copy(inv_v.at[pl.ds(0, _STR)],
                    inv_s.at[bl, pl.ds(sbase, _STR)])
    plsc.subcore_barrier()
    pltpu.sync_copy(inv_s.at[bl], inv_v)

    ch0 = g * _CPG
    for slot in range(2):
      pltpu.async_copy(f_hbm.at[b, ch0, pl.ds(slot * _FCH, _FCH)],
                       fbuf.at[slot], sems.at[slot])
      pltpu.async_copy(idx_s.at[bl, pl.ds(slot * _FCH // 2, _FCH // 2)],
                       ibuf.at[slot], sems.at[2 + slot])

    for cc in range(_CPG):
      ch = ch0 + cc

      def f_pair(kp, _):
        for slot in range(2):
          kc = kp * 2 + slot
          fbase = kc * _FCH
          fbase2 = kc * (_FCH // 2)
          pltpu.make_async_copy(f_hbm.at[b, ch, pl.ds(fbase, _FCH)],
                                fbuf.at[slot], sems.at[slot]).wait()
          pltpu.make_async_copy(
              idx_s.at[bl, pl.ds(fbase2, _FCH // 2)],
              ibuf.at[slot], sems.at[2 + slot]).wait()

          @plsc.parallel_loop(0, _FCH // 32, 1, unroll=8)
          def svec(i):
            o = i * 32
            w = plsc.bitcast(ibuf[slot, pl.ds(i * 16, 16)], jnp.int32)
            iv0 = w & 0xFFFF
            iv1 = lax.shift_right_logical(w, 16)
            plsc.addupdate_scatter(acc_v, [iv0], fbuf[slot, pl.ds(o, 16)])
            plsc.addupdate_scatter(acc_v, [iv1],
                                   fbuf[slot, pl.ds(o + 16, 16)])

          nxt = fbase + 2 * _FCH

          @pl.when(nxt < _N)
          def _():
            pltpu.async_copy(f_hbm.at[b, ch, pl.ds(nxt, _FCH)],
                             fbuf.at[slot], sems.at[slot])
            pltpu.async_copy(idx_s.at[bl, pl.ds(fbase2 + _FCH, _FCH // 2)],
                             ibuf.at[slot], sems.at[2 + slot])
        return 0

      with jax.named_scope(f"scat{cc}"):
        lax.fori_loop(0, _NCH // 2, f_pair, 0)

      if cc + 1 < _CPG:
        for slot in range(2):
          pltpu.async_copy(f_hbm.at[b, ch + 1, pl.ds(slot * _FCH, _FCH)],
                           fbuf.at[slot], sems.at[slot])
          pltpu.async_copy(idx_s.at[bl, pl.ds(slot * _FCH // 2, _FCH // 2)],
                           ibuf.at[slot], sems.at[2 + slot])

      def d_pair(mp, _):
        for slot in range(2):
          m = mp * 2 + slot
          dbase = m * _DCH

          @pl.when(mp >= 1)
          def _():
            pltpu.make_async_copy(dbuf.at[slot],
                                  out_hbm.at[b, ch, pl.ds(dbase, _DCH)],
                                  sems.at[4 + slot]).wait()

          @plsc.parallel_loop(0, _DCH // 16, 1, unroll=8)
          def dvec(j):
            o = j * 16
            a = acc_v[pl.ds(dbase + o, 16)]
            dbuf[slot, pl.ds(o, 16)] = a * inv_v[pl.ds(dbase + o, 16)]
            acc_v[pl.ds(dbase + o, 16)] = zero16
          pltpu.async_copy(dbuf.at[slot],
                           out_hbm.at[b, ch, pl.ds(dbase, _DCH)],
                           sems.at[4 + slot])
        return 0

      with jax.named_scope(f"div{cc}"):
        lax.fori_loop(0, _NDC // 2, d_pair, 0)

      for slot in range(2):
        pltpu.make_async_copy(dbuf.at[slot],
                              out_hbm.at[b, ch, pl.ds(slot * _DCH, _DCH)],
                              sems.at[4 + slot]).wait()

  return vox_kernel


_vox_kernel = _make_sc_kernel()


@jax.jit
def kernel(features, coords):
  out_flat, vox_coords = _vox_kernel(features, coords)
  out = out_flat.reshape(_B, _FC, _R, _R, _R)
  return out, vox_coords


# speedup vs baseline: 1.8099x; 1.0355x over previous
import functools

import jax
import jax.numpy as jnp
from jax import lax
from jax.experimental import pallas as pl
from jax.experimental.pallas import tpu as pltpu
from jax.experimental.pallas import tpu_sc as plsc

_R = 32
_NV = _R * _R * _R
_B, _N = 8, 65536
_FC = 32
_GROUPS = 4
_CPG = _FC // _GROUPS
_QN = _N // _GROUPS
_CCH = 1024
_NQC = _QN // _CCH
_FCH = 4096
_NCH = _N // _FCH
_DCH = 2048
_NDC = _NV // _DCH
_STR = _NV // _GROUPS
_NH = _N // 2


def _make_sc_kernel():
  mesh = plsc.VectorSubcoreMesh(core_axis_name="c", subcore_axis_name="s")

  @functools.partial(
      pl.kernel,
      out_type=(
          jax.ShapeDtypeStruct((_B, _FC, _NV), jnp.float32),
          jax.ShapeDtypeStruct((_B, 3, _N), jnp.int32),
      ),
      mesh=mesh,
      scratch_types=[
          pltpu.VMEM((_NV,), jnp.float32),
          pltpu.VMEM((_NV,), jnp.float32),
          pltpu.VMEM((_NH,), jnp.int32),
          pltpu.VMEM((2, _FCH), jnp.float32),
          pltpu.VMEM((2, _DCH), jnp.float32),
          pltpu.VMEM((2, 3, _CCH), jnp.float32),
          pltpu.VMEM((2, 3, _CCH), jnp.int32),
          pltpu.VMEM((2, _CCH // 2), jnp.int32),
          pltpu.HBM((_B * _GROUPS * _NV,), jnp.float32),
          pltpu.HBM((_B * _NH,), jnp.int32),
          pltpu.SemaphoreType.DMA((12,)),
      ],
      compiler_params=pltpu.CompilerParams(needs_layout_passes=False),
  )
  def vox_kernel(f_hbm, c_hbm, out_hbm, vox_hbm,
                 acc_v, inv_v, idx_v, fbuf, dbuf, cbuf, vbuf, pbuf,
                 part_h, idx_h, sems):
    cid = lax.axis_index("c")
    sid = lax.axis_index("s")
    wid = cid * 16 + sid
    b = wid // _GROUPS
    g = wid % _GROUPS
    hbase = pl.multiple_of(b * _NH, _NH)

    half = jnp.float32(0.5)
    one = jnp.float32(1.0)
    zero16 = jnp.zeros((16,), jnp.float32)
    ones16 = jnp.ones((16,), jnp.float32)

    def axis_round(vals):
      t = jnp.clip(vals * _R, 0.0, _R - 1.0)
      i0 = t.astype(jnp.int32)
      frac = t - i0.astype(jnp.float32)
      up = jnp.where(frac > half, 1, 0) + jnp.where(
          jnp.logical_and(frac == half, (i0 & 1) == 1), 1, 0)
      return i0 + up

    def zero_acc():
      @plsc.parallel_loop(0, _NV // 16, 1, unroll=8)
      def zloop(j):
        acc_v[pl.ds(j * 16, 16)] = zero16

    zero_acc()
    qbase = pl.multiple_of(g * _QN, _QN)
    qbase2 = pl.multiple_of(g * (_QN // 2), _QN // 2)

    for slot in range(2):
      pltpu.async_copy(c_hbm.at[b, :, pl.ds(qbase + slot * _CCH, _CCH)],
                       cbuf.at[slot], sems.at[6 + slot])

    def p1_pair(kp, _):
      for slot in range(2):
        kc = kp * 2 + slot
        base = qbase + kc * _CCH
        base2 = qbase2 + kc * (_CCH // 2)
        pltpu.make_async_copy(c_hbm.at[b, :, pl.ds(base, _CCH)],
                              cbuf.at[slot], sems.at[6 + slot]).wait()

        @pl.when(kp >= 1)
        def _():
          pltpu.make_async_copy(vbuf.at[slot],
                                vox_hbm.at[b, :, pl.ds(base, _CCH)],
                                sems.at[8 + slot]).wait()
          pltpu.make_async_copy(pbuf.at[slot],
                                idx_h.at[pl.ds(hbase + base2, _CCH // 2)],
                                sems.at[10 + slot]).wait()

        @plsc.parallel_loop(0, _CCH // 32, 1, unroll=4)
        def p1v(i):
          off = i * 32
          flats = []
          for u in range(2):
            o = off + u * 16
            vx = axis_round(cbuf[slot, 0, pl.ds(o, 16)])
            vy = axis_round(cbuf[slot, 1, pl.ds(o, 16)])
            vz = axis_round(cbuf[slot, 2, pl.ds(o, 16)])
            vbuf[slot, 0, pl.ds(o, 16)] = vx
            vbuf[slot, 1, pl.ds(o, 16)] = vy
            vbuf[slot, 2, pl.ds(o, 16)] = vz
            flats.append(vx * (_R * _R) + vy * _R + vz)
          pbuf[slot, pl.ds(i * 16, 16)] = flats[0] | (flats[1] << 16)
          plsc.addupdate_scatter(acc_v, [flats[0]], ones16)
          plsc.addupdate_scatter(acc_v, [flats[1]], ones16)

        @pl.when(kc + 2 < _NQC)
        def _():
          pltpu.async_copy(
              c_hbm.at[b, :, pl.ds(qbase + (kc + 2) * _CCH, _CCH)],
              cbuf.at[slot], sems.at[6 + slot])

        pltpu.async_copy(vbuf.at[slot],
                         vox_hbm.at[b, :, pl.ds(base, _CCH)],
                         sems.at[8 + slot])
        pltpu.async_copy(pbuf.at[slot],
                         idx_h.at[pl.ds(hbase + base2, _CCH // 2)],
                         sems.at[10 + slot])
      return 0

    with jax.named_scope("p1"):
      lax.fori_loop(0, _NQC // 2, p1_pair, 0)

    for slot in range(2):
      pltpu.make_async_copy(vbuf.at[slot],
                            vox_hbm.at[b, :, pl.ds(slot * _CCH, _CCH)],
                            sems.at[8 + slot]).wait()
      pltpu.make_async_copy(pbuf.at[slot],
                            idx_h.at[pl.ds(slot * (_CCH // 2), _CCH // 2)],
                            sems.at[10 + slot]).wait()

    with jax.named_scope("pc"):
      pltpu.sync_copy(
          acc_v,
          part_h.at[pl.ds(
              pl.multiple_of((b * _GROUPS + g) * _NV, _NV), _NV)])
      zero_acc()
    plsc.subcore_barrier()

    with jax.named_scope("mrg"):
      sbase = pl.multiple_of(g * _STR, _STR)
      for j in range(_GROUPS):
        pltpu.sync_copy(
            part_h.at[pl.ds(
                pl.multiple_of((b * _GROUPS + j) * _NV + sbase, _STR),
                _STR)],
            inv_v.at[pl.ds(j * _STR, _STR)])

      @plsc.parallel_loop(0, _STR // 16, 1, unroll=4)
      def mloop(t):
        o = t * 16
        v = (inv_v[pl.ds(o, 16)] + inv_v[pl.ds(_STR + o, 16)]
             + inv_v[pl.ds(2 * _STR + o, 16)]
             + inv_v[pl.ds(3 * _STR + o, 16)])
        inv_v[pl.ds(o, 16)] = one / jnp.maximum(v, one)

      pltpu.sync_copy(
          inv_v.at[pl.ds(0, _STR)],
          part_h.at[pl.ds(
              pl.multiple_of(b * _GROUPS * _NV + sbase, _STR), _STR)])
    plsc.subcore_barrier()

    with jax.named_scope("ld"):
      pltpu.sync_copy(
          part_h.at[pl.ds(pl.multiple_of(b * _GROUPS * _NV, _NV), _NV)],
          inv_v)
      pltpu.sync_copy(idx_h.at[pl.ds(hbase, _NH)], idx_v)

    ch0 = g * _CPG
    for slot in range(2):
      pltpu.async_copy(f_hbm.at[b, ch0, pl.ds(slot * _FCH, _FCH)],
                       fbuf.at[slot], sems.at[slot])

    for cc in range(_CPG):
      ch = ch0 + cc

      def f_pair(kp, _):
        for slot in range(2):
          kc = kp * 2 + slot
          fbase = kc * _FCH
          fbase2 = kc * (_FCH // 2)
          pltpu.make_async_copy(f_hbm.at[b, ch, pl.ds(fbase, _FCH)],
                                fbuf.at[slot], sems.at[slot]).wait()

          @plsc.parallel_loop(0, _FCH // 32, 1, unroll=8)
          def svec(i):
            o = i * 32
            w = idx_v[pl.ds(fbase2 + i * 16, 16)]
            iv0 = w & 0xFFFF
            iv1 = lax.shift_right_logical(w, 16)
            plsc.addupdate_scatter(acc_v, [iv0], fbuf[slot, pl.ds(o, 16)])
            plsc.addupdate_scatter(acc_v, [iv1],
                                   fbuf[slot, pl.ds(o + 16, 16)])

          nxt = fbase + 2 * _FCH

          @pl.when(nxt < _N)
          def _():
            pltpu.async_copy(f_hbm.at[b, ch, pl.ds(nxt, _FCH)],
                             fbuf.at[slot], sems.at[slot])
        return 0

      with jax.named_scope(f"scat{cc}"):
        lax.fori_loop(0, _NCH // 2, f_pair, 0)

      if cc + 1 < _CPG:
        for slot in range(2):
          pltpu.async_copy(f_hbm.at[b, ch + 1, pl.ds(slot * _FCH, _FCH)],
                           fbuf.at[slot], sems.at[slot])

      def d_pair(mp, _):
        for slot in range(2):
          m = mp * 2 + slot
          dbase = m * _DCH

          @pl.when(mp >= 1)
          def _():
            pltpu.make_async_copy(dbuf.at[slot],
                                  out_hbm.at[b, ch, pl.ds(dbase, _DCH)],
                                  sems.at[4 + slot]).wait()

          @plsc.parallel_loop(0, _DCH // 16, 1, unroll=8)
          def dvec(j):
            o = j * 16
            a = acc_v[pl.ds(dbase + o, 16)]
            dbuf[slot, pl.ds(o, 16)] = a * inv_v[pl.ds(dbase + o, 16)]
            acc_v[pl.ds(dbase + o, 16)] = zero16

          pltpu.async_copy(dbuf.at[slot],
                           out_hbm.at[b, ch, pl.ds(dbase, _DCH)],
                           sems.at[4 + slot])
        return 0

      with jax.named_scope(f"div{cc}"):
        lax.fori_loop(0, _NDC // 2, d_pair, 0)

      for slot in range(2):
        pltpu.make_async_copy(dbuf.at[slot],
                              out_hbm.at[b, ch, pl.ds(slot * _DCH, _DCH)],
                              sems.at[4 + slot]).wait()

  return vox_kernel


_vox_kernel = _make_sc_kernel()


@jax.jit
def kernel(features, coords):
  out_flat, vox_coords = _vox_kernel(features, coords)
  out = out_flat.reshape(_B, _FC, _R, _R, _R)
  return out, vox_coords


# speedup vs baseline: 2.1673x; 1.1975x over previous
import functools

import jax
import jax.numpy as jnp
from jax import lax
from jax.experimental import pallas as pl
from jax.experimental.pallas import tpu as pltpu
from jax.experimental.pallas import tpu_sc as plsc

_R = 32
_NV = _R * _R * _R
_B, _N = 8, 65536
_FC = 32
_GROUPS = 4
_CPG = _FC // _GROUPS
_QN = _N // _GROUPS
_CCH = 512
_NQC = _QN // _CCH
_FCH = 4096
_NCH = _N // _FCH
_DCH = 2048
_NDC = _NV // _DCH
_STR = _NV // _GROUPS
_NH = _N // 2


def _make_sc_kernel():
  mesh = plsc.VectorSubcoreMesh(core_axis_name="c", subcore_axis_name="s")

  @functools.partial(
      pl.kernel,
      out_type=(
          jax.ShapeDtypeStruct((_B, _FC, _R, _R, _R), jnp.float32),
          jax.ShapeDtypeStruct((_B, 3, _N), jnp.int32),
      ),
      mesh=mesh,
      scratch_types=[
          pltpu.VMEM((_NV,), jnp.float32),
          pltpu.VMEM((_NV,), jnp.float32),
          pltpu.VMEM((_NH,), jnp.int32),
          pltpu.VMEM((2, _FCH), jnp.float32),
          pltpu.VMEM((2, _R, _R), jnp.float32),
          pltpu.VMEM((2, 3, _CCH), jnp.float32),
          pltpu.VMEM((2, 3, _CCH), jnp.int32),
          pltpu.VMEM((2, _CCH // 2), jnp.int32),
          pltpu.HBM((_B * _GROUPS * _NV,), jnp.float32),
          pltpu.HBM((_B * _NH,), jnp.int32),
          pltpu.SemaphoreType.DMA((12,)),
      ],
      compiler_params=pltpu.CompilerParams(needs_layout_passes=False),
  )
  def vox_kernel(f_hbm, c_hbm, out_hbm, vox_hbm,
                 acc_v, inv_v, idx_v, fbuf, dbuf, cbuf, vbuf, pbuf,
                 part_h, idx_h, sems):
    cid = lax.axis_index("c")
    sid = lax.axis_index("s")
    wid = cid * 16 + sid
    b = wid // _GROUPS
    g = wid % _GROUPS
    hbase = pl.multiple_of(b * _NH, _NH)

    half = jnp.float32(0.5)
    one = jnp.float32(1.0)
    zero16 = jnp.zeros((16,), jnp.float32)
    ones16 = jnp.ones((16,), jnp.float32)

    def axis_round(vals):
      t = jnp.clip(vals * _R, 0.0, _R - 1.0)
      i0 = t.astype(jnp.int32)
      frac = t - i0.astype(jnp.float32)
      up = jnp.where(frac > half, 1, 0) + jnp.where(
          jnp.logical_and(frac == half, (i0 & 1) == 1), 1, 0)
      return i0 + up

    def zero_acc():
      @plsc.parallel_loop(0, _NV // 16, 1, unroll=8)
      def zloop(j):
        acc_v[pl.ds(j * 16, 16)] = zero16

    zero_acc()
    qbase = pl.multiple_of(g * _QN, _QN)
    qbase2 = pl.multiple_of(g * (_QN // 2), _QN // 2)

    for slot in range(2):
      pltpu.async_copy(c_hbm.at[b, :, pl.ds(qbase + slot * _CCH, _CCH)],
                       cbuf.at[slot], sems.at[6 + slot])

    def p1_pair(kp, _):
      for slot in range(2):
        kc = kp * 2 + slot
        base = qbase + kc * _CCH
        base2 = qbase2 + kc * (_CCH // 2)
        pltpu.make_async_copy(c_hbm.at[b, :, pl.ds(base, _CCH)],
                              cbuf.at[slot], sems.at[6 + slot]).wait()

        @pl.when(kp >= 1)
        def _():
          pltpu.make_async_copy(vbuf.at[slot],
                                vox_hbm.at[b, :, pl.ds(base, _CCH)],
                                sems.at[8 + slot]).wait()
          pltpu.make_async_copy(pbuf.at[slot],
                                idx_h.at[pl.ds(hbase + base2, _CCH // 2)],
                                sems.at[10 + slot]).wait()

        @plsc.parallel_loop(0, _CCH // 32, 1, unroll=4)
        def p1v(i):
          off = i * 32
          flats = []
          for u in range(2):
            o = off + u * 16
            vx = axis_round(cbuf[slot, 0, pl.ds(o, 16)])
            vy = axis_round(cbuf[slot, 1, pl.ds(o, 16)])
            vz = axis_round(cbuf[slot, 2, pl.ds(o, 16)])
            vbuf[slot, 0, pl.ds(o, 16)] = vx
            vbuf[slot, 1, pl.ds(o, 16)] = vy
            vbuf[slot, 2, pl.ds(o, 16)] = vz
            flats.append(vx * (_R * _R) + vy * _R + vz)
          pbuf[slot, pl.ds(i * 16, 16)] = flats[0] | (flats[1] << 16)
          plsc.addupdate_scatter(acc_v, [flats[0]], ones16)
          plsc.addupdate_scatter(acc_v, [flats[1]], ones16)

        @pl.when(kc + 2 < _NQC)
        def _():
          pltpu.async_copy(
              c_hbm.at[b, :, pl.ds(qbase + (kc + 2) * _CCH, _CCH)],
              cbuf.at[slot], sems.at[6 + slot])

        pltpu.async_copy(vbuf.at[slot],
                         vox_hbm.at[b, :, pl.ds(base, _CCH)],
                         sems.at[8 + slot])
        pltpu.async_copy(pbuf.at[slot],
                         idx_h.at[pl.ds(hbase + base2, _CCH // 2)],
                         sems.at[10 + slot])
      return 0

    with jax.named_scope("p1"):
      lax.fori_loop(0, _NQC // 2, p1_pair, 0)

    for slot in range(2):
      pltpu.make_async_copy(vbuf.at[slot],
                            vox_hbm.at[b, :, pl.ds(slot * _CCH, _CCH)],
                            sems.at[8 + slot]).wait()
      pltpu.make_async_copy(pbuf.at[slot],
                            idx_h.at[pl.ds(slot * (_CCH // 2), _CCH // 2)],
                            sems.at[10 + slot]).wait()

    with jax.named_scope("pc"):
      pltpu.sync_copy(
          acc_v,
          part_h.at[pl.ds(
              pl.multiple_of((b * _GROUPS + g) * _NV, _NV), _NV)])
      zero_acc()
    plsc.subcore_barrier()

    with jax.named_scope("mrg"):
      sbase = pl.multiple_of(g * _STR, _STR)
      for j in range(_GROUPS):
        pltpu.sync_copy(
            part_h.at[pl.ds(
                pl.multiple_of((b * _GROUPS + j) * _NV + sbase, _STR),
                _STR)],
            inv_v.at[pl.ds(j * _STR, _STR)])

      @plsc.parallel_loop(0, _STR // 16, 1, unroll=4)
      def mloop(t):
        o = t * 16
        v = (inv_v[pl.ds(o, 16)] + inv_v[pl.ds(_STR + o, 16)]
             + inv_v[pl.ds(2 * _STR + o, 16)]
             + inv_v[pl.ds(3 * _STR + o, 16)])
        inv_v[pl.ds(o, 16)] = one / jnp.maximum(v, one)

      pltpu.sync_copy(
          inv_v.at[pl.ds(0, _STR)],
          part_h.at[pl.ds(
              pl.multiple_of(b * _GROUPS * _NV + sbase, _STR), _STR)])
    plsc.subcore_barrier()

    with jax.named_scope("ld"):
      pltpu.sync_copy(
          part_h.at[pl.ds(pl.multiple_of(b * _GROUPS * _NV, _NV), _NV)],
          inv_v)
      pltpu.sync_copy(idx_h.at[pl.ds(hbase, _NH)], idx_v)

    ch0 = g * _CPG
    for slot in range(2):
      pltpu.async_copy(f_hbm.at[b, ch0, pl.ds(slot * _FCH, _FCH)],
                       fbuf.at[slot], sems.at[slot])

    for cc in range(_CPG):
      ch = ch0 + cc

      def f_pair(kp, _):
        for slot in range(2):
          kc = kp * 2 + slot
          fbase = kc * _FCH
          fbase2 = kc * (_FCH // 2)
          pltpu.make_async_copy(f_hbm.at[b, ch, pl.ds(fbase, _FCH)],
                                fbuf.at[slot], sems.at[slot]).wait()

          @plsc.parallel_loop(0, _FCH // 32, 1, unroll=8)
          def svec(i):
            o = i * 32
            w = idx_v[pl.ds(fbase2 + i * 16, 16)]
            iv0 = w & 0xFFFF
            iv1 = lax.shift_right_logical(w, 16)
            plsc.addupdate_scatter(acc_v, [iv0], fbuf[slot, pl.ds(o, 16)])
            plsc.addupdate_scatter(acc_v, [iv1],
                                   fbuf[slot, pl.ds(o + 16, 16)])

          nxt = fbase + 2 * _FCH

          @pl.when(nxt < _N)
          def _():
            pltpu.async_copy(f_hbm.at[b, ch, pl.ds(nxt, _FCH)],
                             fbuf.at[slot], sems.at[slot])
        return 0

      with jax.named_scope(f"scat{cc}"):
        lax.fori_loop(0, _NCH // 2, f_pair, 0)

      if cc + 1 < _CPG:
        for slot in range(2):
          pltpu.async_copy(f_hbm.at[b, ch + 1, pl.ds(slot * _FCH, _FCH)],
                           fbuf.at[slot], sems.at[slot])

      def d_pair(mp, _):
        for slot in range(2):
          m = mp * 2 + slot
          dbase = m * 1024

          @pl.when(mp >= 1)
          def _():
            pltpu.make_async_copy(dbuf.at[slot], out_hbm.at[b, ch, m],
                                  sems.at[4 + slot]).wait()

          @plsc.parallel_loop(0, _R, 1, unroll=8)
          def dvec(y):
            for u in range(2):
              o = y * _R + u * 16
              a = acc_v[pl.ds(dbase + o, 16)]
              dbuf[slot, y, pl.ds(u * 16, 16)] = (
                  a * inv_v[pl.ds(dbase + o, 16)])
              acc_v[pl.ds(dbase + o, 16)] = zero16

          pltpu.async_copy(dbuf.at[slot], out_hbm.at[b, ch, m],
                           sems.at[4 + slot])
        return 0

      with jax.named_scope(f"div{cc}"):
        lax.fori_loop(0, _R // 2, d_pair, 0)

      for slot in range(2):
        pltpu.make_async_copy(dbuf.at[slot], out_hbm.at[b, ch, slot],
                              sems.at[4 + slot]).wait()

  return vox_kernel


_vox_kernel = _make_sc_kernel()


@jax.jit
def kernel(features, coords):
  return _vox_kernel(features, coords)
